# baseline (speedup 1.0000x reference)
import functools

import jax
import jax.numpy as jnp
from jax import lax
from jax.experimental import pallas as pl
from jax.experimental.pallas import tpu as pltpu
import jax.experimental.pallas.tpu_sc as plsc

_NC = 2
_NS = 16
_NW = _NC * _NS
_CH = 128
_INTERPRET = False


def _sc_ef_body(NCH, efr, dstr, z_f, f_out, facc, dst_v, ef_v):
    c = lax.axis_index("c")
    s = lax.axis_index("s")
    wid = c * _NS + s
    zrows = facc.shape[0] // _NS
    pltpu.sync_copy(z_f, facc.at[pl.ds(s * zrows, zrows)])
    plsc.subcore_barrier()

    def body(j, carry):
        pltpu.sync_copy(dstr.at[wid, j], dst_v)
        pltpu.sync_copy(efr.at[wid, j], ef_v)
        pltpu.sync_copy(ef_v, facc.at[dst_v], add=True)
        return carry

    lax.fori_loop(0, NCH, body, 0)
    plsc.subcore_barrier()
    pltpu.sync_copy(facc.at[pl.ds(s * zrows, zrows)],
                    f_out.at[c, pl.ds(s * zrows, zrows)])


def _sc_seg_body(NCH, x_hbm, srcr, dstr, z_d, a_out,
                 acc, src_v0, dst_v0, rows_v0, sem0,
                 src_v1, dst_v1, rows_v1, sem1):
    c = lax.axis_index("c")
    s = lax.axis_index("s")
    wid = c * _NS + s
    zrows = acc.shape[0] // _NS
    pltpu.sync_copy(z_d, acc.at[pl.ds(s * zrows, zrows)])
    plsc.subcore_barrier()

    bufs = ((src_v0, dst_v0, rows_v0, sem0), (src_v1, dst_v1, rows_v1, sem1))
    for b in range(2):
        sv, dv, rv, sm = bufs[b]
        pltpu.sync_copy(srcr.at[wid, b], sv)
        pltpu.sync_copy(dstr.at[wid, b], dv)
        pltpu.async_copy(x_hbm.at[sv], rv, sm)

    def body(g, carry):
        for b in range(2):
            sv, dv, rv, sm = bufs[b]
            j = 2 * g + b
            pltpu.make_async_copy(x_hbm.at[sv], rv, sm).wait()
            pltpu.sync_copy(rv, acc.at[dv], add=True)

            @pl.when(j + 2 < NCH)
            def _():
                pltpu.sync_copy(srcr.at[wid, j + 2], sv)
                pltpu.sync_copy(dstr.at[wid, j + 2], dv)
                pltpu.async_copy(x_hbm.at[sv], rv, sm)

        return carry

    lax.fori_loop(0, NCH // 2, body, 0)
    plsc.subcore_barrier()
    pltpu.sync_copy(acc.at[pl.ds(s * zrows, zrows)],
                    a_out.at[c, pl.ds(s * zrows, zrows)])


def _make_sc_ef(NP, NCH, DEA):
    mesh = plsc.VectorSubcoreMesh(core_axis_name="c", subcore_axis_name="s",
                                  num_cores=_NC, num_subcores=_NS)
    return pl.kernel(
        functools.partial(_sc_ef_body, NCH),
        out_type=jax.ShapeDtypeStruct((_NC, NP, DEA), jnp.float32),
        mesh=mesh,
        scratch_types=[
            pltpu.VMEM_SHARED((NP, DEA), jnp.float32),
            pltpu.VMEM((_CH,), jnp.int32),
            pltpu.VMEM((_CH, DEA), jnp.float32),
        ],
        compiler_params=pltpu.CompilerParams(use_tc_tiling_on_sc=False),
        interpret=_INTERPRET,
    )


def _make_sc_seg(NP, NCH, D):
    mesh = plsc.VectorSubcoreMesh(core_axis_name="c", subcore_axis_name="s",
                                  num_cores=_NC, num_subcores=_NS)
    return pl.kernel(
        functools.partial(_sc_seg_body, NCH),
        out_type=jax.ShapeDtypeStruct((_NC, NP, D), jnp.float32),
        mesh=mesh,
        scratch_types=[
            pltpu.VMEM_SHARED((NP, D), jnp.float32),
            pltpu.VMEM((_CH,), jnp.int32),
            pltpu.VMEM((_CH,), jnp.int32),
            pltpu.VMEM((_CH, D), jnp.float32),
            pltpu.SemaphoreType.DMA,
            pltpu.VMEM((_CH,), jnp.int32),
            pltpu.VMEM((_CH,), jnp.int32),
            pltpu.VMEM((_CH, D), jnp.float32),
            pltpu.SemaphoreType.DMA,
        ],
        interpret=_INTERPRET,
    )


def _tc1_body(a_ref, f_ref, w1n_ref, w1ea_ref, w2ea_ref, x1_ref, efw2_ref):
    a = a_ref[0] + a_ref[1]
    f = f_ref[0] + f_ref[1]
    x1 = (jnp.dot(a, w1n_ref[...], preferred_element_type=jnp.float32, precision=lax.Precision.HIGHEST)
          + jnp.dot(f, w1ea_ref[...], preferred_element_type=jnp.float32, precision=lax.Precision.HIGHEST))
    x1_ref[...] = jnp.maximum(x1, 0.0)
    efw2_ref[...] = jnp.dot(f, w2ea_ref[...], preferred_element_type=jnp.float32, precision=lax.Precision.HIGHEST)


def _make_tc1(NP, D, DEA, H):
    BM = NP // 8
    grid = (8,)
    return pl.pallas_call(
        _tc1_body,
        grid=grid,
        in_specs=[
            pl.BlockSpec((_NC, BM, D), lambda i: (0, i, 0)),
            pl.BlockSpec((_NC, BM, DEA), lambda i: (0, i, 0)),
            pl.BlockSpec((D, H), lambda i: (0, 0)),
            pl.BlockSpec((DEA, H), lambda i: (0, 0)),
            pl.BlockSpec((DEA, H), lambda i: (0, 0)),
        ],
        out_specs=[
            pl.BlockSpec((BM, H), lambda i: (i, 0)),
            pl.BlockSpec((BM, H), lambda i: (i, 0)),
        ],
        out_shape=[jax.ShapeDtypeStruct((NP, H), jnp.float32),
                   jax.ShapeDtypeStruct((NP, H), jnp.float32)],
        interpret=_INTERPRET,
    )


def _tc2_body(N, BM, b_ref, efw2_ref, w2n_ref, out_ref):
    i = pl.program_id(0)
    b = b_ref[0] + b_ref[1]
    out2 = (jnp.dot(b, w2n_ref[...], preferred_element_type=jnp.float32, precision=lax.Precision.HIGHEST)
            + efw2_ref[...])
    rmax = jnp.max(out2, axis=1)
    rmin = jnp.min(out2, axis=1)
    rows = lax.broadcasted_iota(jnp.int32, (BM,), 0) + i * BM
    m = ((rmax != rmin) & (rows < N)).astype(jnp.float32)
    part = jnp.sum(out2 * m[:, None], axis=0)

    @pl.when(i == 0)
    def _():
        out_ref[...] = jnp.zeros_like(out_ref)

    out_ref[...] += part[None, :]


def _make_tc2(N, NP, D, H):
    BM = NP // 8
    grid = (8,)
    return pl.pallas_call(
        functools.partial(_tc2_body, N, BM),
        grid=grid,
        in_specs=[
            pl.BlockSpec((_NC, BM, D), lambda i: (0, i, 0)),
            pl.BlockSpec((BM, H), lambda i: (i, 0)),
            pl.BlockSpec((D, H), lambda i: (0, 0)),
        ],
        out_specs=pl.BlockSpec((1, H), lambda i: (0, 0)),
        out_shape=jax.ShapeDtypeStruct((1, H), jnp.float32),
        interpret=_INTERPRET,
    )


def kernel(node_feature, edge_index, edge_feature, W1n, b1n, W1e, b1e,
           W2n, b2n, W2e, b2e):
    N, D = node_feature.shape
    E, DE = edge_feature.shape
    H = W1n.shape[1]
    DEA = 32

    src = edge_index[0].astype(jnp.int32)
    dst = edge_index[1].astype(jnp.int32)

    NP = ((N + 1 + 127) // 128) * 128
    epw = _NW * _CH * 2
    Ep = ((E + epw - 1) // epw) * epw
    pad = Ep - E
    pad_dst = N + jnp.arange(pad, dtype=jnp.int32) % (NP - N)
    src = jnp.concatenate([src, jnp.zeros((pad,), jnp.int32)])
    dst = jnp.concatenate([dst, pad_dst])
    NCH = Ep // (_NW * _CH)
    srcr = src.reshape(_NW, NCH, _CH)
    dstr = dst.reshape(_NW, NCH, _CH)

    ef_aug = (jnp.zeros((Ep, DEA), jnp.float32)
              .at[:E, :DE].set(edge_feature)
              .at[:E, DE].set(1.0))
    efr = ef_aug.reshape(_NW, NCH, _CH, DEA)

    zrows = NP // _NS
    z_d = jnp.zeros((zrows, D), jnp.float32)
    z_f = jnp.zeros((zrows, DEA), jnp.float32)

    W1ea = jnp.zeros((DEA, H), jnp.float32).at[:DE].set(W1e).at[DE].set(b1n + b1e)
    W2ea = jnp.zeros((DEA, H), jnp.float32).at[:DE].set(W2e).at[DE].set(b2n + b2e)

    F = _make_sc_ef(NP, NCH, DEA)(efr, dstr, z_f)
    A = _make_sc_seg(NP, NCH, D)(node_feature, srcr, dstr, z_d)
    X1, EFW2 = _make_tc1(NP, D, DEA, H)(A, F, W1n, W1ea, W2ea)
    B = _make_sc_seg(NP, NCH, H)(X1, srcr, dstr, z_d)
    pooled = _make_tc2(N, NP, H, H)(B, EFW2, W2n)
    return pooled.reshape(H)


# speedup vs baseline: 1.5743x; 1.5743x over previous
import functools

import jax
import jax.numpy as jnp
from jax import lax
from jax.experimental import pallas as pl
from jax.experimental.pallas import tpu as pltpu
import jax.experimental.pallas.tpu_sc as plsc

_NC = 2
_NS = 16
_CH = 128
_INTERPRET = False


def _mesh():
    return plsc.VectorSubcoreMesh(core_axis_name="c", subcore_axis_name="s",
                                  num_cores=_NC, num_subcores=_NS)


def _sc_seg_body(NCH, with_ef, args):
    if with_ef:
        (xh_hbm, srcr, dstr, efr, z_h, z_f, a_out, f_out, acc, facc, xs,
         sv0, dv0, rv0, sm0, sv1, dv1, rv1, sm1, ef_v) = args
    else:
        (xh_hbm, srcr, dstr, z_h, a_out, acc, xs,
         sv0, dv0, rv0, sm0, sv1, dv1, rv1, sm1) = args
    c = lax.axis_index("c")
    s = lax.axis_index("s")
    zr = acc.shape[0] // _NS
    pltpu.sync_copy(z_h, acc.at[pl.ds(s * zr, zr)])
    pltpu.sync_copy(xh_hbm.at[c, pl.ds(s * zr, zr)], xs.at[pl.ds(s * zr, zr)])
    if with_ef:
        pltpu.sync_copy(z_f, facc.at[pl.ds(s * zr, zr)])
    plsc.subcore_barrier()

    bufs = ((sv0, dv0, rv0, sm0), (sv1, dv1, rv1, sm1))
    for b in range(2):
        sv, dv, rv, sm = bufs[b]
        pltpu.sync_copy(srcr.at[s, b], sv)
        pltpu.sync_copy(dstr.at[s, b], dv)
        pltpu.async_copy(xs.at[sv], rv, sm)

    def body(g, carry):
        for b in range(2):
            sv, dv, rv, sm = bufs[b]
            j = 2 * g + b
            pltpu.make_async_copy(xs.at[sv], rv, sm).wait()
            pltpu.sync_copy(rv, acc.at[dv], add=True)
            if with_ef:
                @pl.when(c == b)
                def _():
                    pltpu.sync_copy(efr.at[s, j], ef_v)
                    pltpu.sync_copy(ef_v, facc.at[dv], add=True)

            @pl.when(j + 2 < NCH)
            def _():
                pltpu.sync_copy(srcr.at[s, j + 2], sv)
                pltpu.sync_copy(dstr.at[s, j + 2], dv)
                pltpu.async_copy(xs.at[sv], rv, sm)

        return carry

    lax.fori_loop(0, NCH // 2, body, 0)
    plsc.subcore_barrier()
    pltpu.sync_copy(acc.at[pl.ds(s * zr, zr)], a_out.at[c, pl.ds(s * zr, zr)])
    if with_ef:
        pltpu.sync_copy(facc.at[pl.ds(s * zr, zr)],
                        f_out.at[c, pl.ds(s * zr, zr)])


def _make_sc_seg1(NP, NCH, DH, DEA):
    def body(*args):
        _sc_seg_body(NCH, True, args)

    return pl.kernel(
        body,
        out_type=[jax.ShapeDtypeStruct((_NC, NP, DH), jnp.float32),
                  jax.ShapeDtypeStruct((_NC, NP, DEA), jnp.float32)],
        mesh=_mesh(),
        scratch_types=[
            pltpu.VMEM_SHARED((NP, DH), jnp.float32),
            pltpu.VMEM_SHARED((NP, DEA), jnp.float32),
            pltpu.VMEM_SHARED((NP, DH), jnp.float32),
            pltpu.VMEM((_CH,), jnp.int32),
            pltpu.VMEM((_CH,), jnp.int32),
            pltpu.VMEM((_CH, DH), jnp.float32),
            pltpu.SemaphoreType.DMA,
            pltpu.VMEM((_CH,), jnp.int32),
            pltpu.VMEM((_CH,), jnp.int32),
            pltpu.VMEM((_CH, DH), jnp.float32),
            pltpu.SemaphoreType.DMA,
            pltpu.VMEM((_CH, DEA), jnp.float32),
        ],
        compiler_params=pltpu.CompilerParams(use_tc_tiling_on_sc=False),
        interpret=_INTERPRET,
    )


def _make_sc_seg2(NP, NCH, DH):
    def body(*args):
        _sc_seg_body(NCH, False, args)

    return pl.kernel(
        body,
        out_type=jax.ShapeDtypeStruct((_NC, NP, DH), jnp.float32),
        mesh=_mesh(),
        scratch_types=[
            pltpu.VMEM_SHARED((NP, DH), jnp.float32),
            pltpu.VMEM_SHARED((NP, DH), jnp.float32),
            pltpu.VMEM((_CH,), jnp.int32),
            pltpu.VMEM((_CH,), jnp.int32),
            pltpu.VMEM((_CH, DH), jnp.float32),
            pltpu.SemaphoreType.DMA,
            pltpu.VMEM((_CH,), jnp.int32),
            pltpu.VMEM((_CH,), jnp.int32),
            pltpu.VMEM((_CH, DH), jnp.float32),
            pltpu.SemaphoreType.DMA,
        ],
        compiler_params=pltpu.CompilerParams(use_tc_tiling_on_sc=False),
        interpret=_INTERPRET,
    )


def _tc1_body(a_ref, f_ref, w1n_ref, w1ea_ref, w2ea_ref, x1_ref, efw2_ref):
    a = jnp.concatenate([a_ref[0], a_ref[1]], axis=-1)
    f = f_ref[0] + f_ref[1]
    x1 = (jnp.dot(a, w1n_ref[...], preferred_element_type=jnp.float32,
                  precision=lax.Precision.HIGHEST)
          + jnp.dot(f, w1ea_ref[...], preferred_element_type=jnp.float32,
                    precision=lax.Precision.HIGHEST))
    x1 = jnp.maximum(x1, 0.0)
    DH = x1_ref.shape[-1]
    x1_ref[0] = x1[:, :DH]
    x1_ref[1] = x1[:, DH:]
    efw2_ref[...] = jnp.dot(f, w2ea_ref[...], preferred_element_type=jnp.float32,
                            precision=lax.Precision.HIGHEST)


def _make_tc1(NP, D, DEA, H):
    BM = NP // 8
    DH = D // 2
    grid = (8,)
    return pl.pallas_call(
        _tc1_body,
        grid=grid,
        in_specs=[
            pl.BlockSpec((_NC, BM, DH), lambda i: (0, i, 0)),
            pl.BlockSpec((_NC, BM, DEA), lambda i: (0, i, 0)),
            pl.BlockSpec((D, H), lambda i: (0, 0)),
            pl.BlockSpec((DEA, H), lambda i: (0, 0)),
            pl.BlockSpec((DEA, H), lambda i: (0, 0)),
        ],
        out_specs=[
            pl.BlockSpec((_NC, BM, H // 2), lambda i: (0, i, 0)),
            pl.BlockSpec((BM, H), lambda i: (i, 0)),
        ],
        out_shape=[jax.ShapeDtypeStruct((_NC, NP, H // 2), jnp.float32),
                   jax.ShapeDtypeStruct((NP, H), jnp.float32)],
        interpret=_INTERPRET,
    )


def _tc2_body(N, BM, b_ref, efw2_ref, w2n_ref, out_ref):
    i = pl.program_id(0)
    b = jnp.concatenate([b_ref[0], b_ref[1]], axis=-1)
    out2 = (jnp.dot(b, w2n_ref[...], preferred_element_type=jnp.float32,
                    precision=lax.Precision.HIGHEST)
            + efw2_ref[...])
    rmax = jnp.max(out2, axis=1)
    rmin = jnp.min(out2, axis=1)
    rows = lax.broadcasted_iota(jnp.int32, (BM,), 0) + i * BM
    m = ((rmax != rmin) & (rows < N)).astype(jnp.float32)
    part = jnp.sum(out2 * m[:, None], axis=0)

    @pl.when(i == 0)
    def _():
        out_ref[...] = jnp.zeros_like(out_ref)

    out_ref[...] += part[None, :]


def _make_tc2(N, NP, H):
    BM = NP // 8
    grid = (8,)
    return pl.pallas_call(
        functools.partial(_tc2_body, N, BM),
        grid=grid,
        in_specs=[
            pl.BlockSpec((_NC, BM, H // 2), lambda i: (0, i, 0)),
            pl.BlockSpec((BM, H), lambda i: (i, 0)),
            pl.BlockSpec((H, H), lambda i: (0, 0)),
        ],
        out_specs=pl.BlockSpec((1, H), lambda i: (0, 0)),
        out_shape=jax.ShapeDtypeStruct((1, H), jnp.float32),
        interpret=_INTERPRET,
    )


def kernel(node_feature, edge_index, edge_feature, W1n, b1n, W1e, b1e,
           W2n, b2n, W2e, b2e):
    N, D = node_feature.shape
    E, DE = edge_feature.shape
    H = W1n.shape[1]
    DH = D // 2
    DEA = 32

    src = edge_index[0].astype(jnp.int32)
    dst = edge_index[1].astype(jnp.int32)

    NP = ((N + 1 + 127) // 128) * 128
    epw = _NS * _CH * 2
    Ep = ((E + epw - 1) // epw) * epw
    pad = Ep - E
    pad_dst = N + jnp.arange(pad, dtype=jnp.int32) % (NP - N)
    src = jnp.concatenate([src, jnp.zeros((pad,), jnp.int32)])
    dst = jnp.concatenate([dst, pad_dst])
    NCH = Ep // (_NS * _CH)
    srcr = src.reshape(_NS, NCH, _CH)
    dstr = dst.reshape(_NS, NCH, _CH)

    ef_aug = (jnp.zeros((Ep, DEA), jnp.float32)
              .at[:E, :DE].set(edge_feature)
              .at[:E, DE].set(1.0))
    efr = ef_aug.reshape(_NS, NCH, _CH, DEA)

    zrows = NP // _NS
    z_h = jnp.zeros((zrows, DH), jnp.float32)
    z_f = jnp.zeros((zrows, DEA), jnp.float32)

    xpad = jnp.zeros((NP, D), jnp.float32).at[:N].set(node_feature)
    xh = jnp.stack([xpad[:, :DH], xpad[:, DH:]])

    W1ea = jnp.zeros((DEA, H), jnp.float32).at[:DE].set(W1e).at[DE].set(b1n + b1e)
    W2ea = jnp.zeros((DEA, H), jnp.float32).at[:DE].set(W2e).at[DE].set(b2n + b2e)

    A, F = _make_sc_seg1(NP, NCH, DH, DEA)(xh, srcr, dstr, efr, z_h, z_f)
    X1h, EFW2 = _make_tc1(NP, D, DEA, H)(A, F, W1n, W1ea, W2ea)
    B = _make_sc_seg2(NP, NCH, H // 2)(X1h, srcr, dstr, z_h)
    pooled = _make_tc2(N, NP, H)(B, EFW2, W2n)
    return pooled.reshape(H)


# speedup vs baseline: 2.3469x; 1.4908x over previous
import functools

import jax
import jax.numpy as jnp
from jax import lax
from jax.experimental import pallas as pl
from jax.experimental.pallas import tpu as pltpu
import jax.experimental.pallas.tpu_sc as plsc

_NC = 2
_NS = 16
_CH = 128
_INTERPRET = False


def _mesh():
    return plsc.VectorSubcoreMesh(core_axis_name="c", subcore_axis_name="s",
                                  num_cores=_NC, num_subcores=_NS)


def _sc_seg_body(NCH, NB, with_ef, args):
    if with_ef:
        (xh_hbm, sdr, efr, z_h, z_f, a_out, f_out,
         acc, facc, xs, ef_v, efsm) = args[:12]
        bufs = args[12:]
    else:
        (xh_hbm, sdr, z_h, a_out, acc, xs) = args[:6]
        bufs = args[6:]
    c = lax.axis_index("c")
    s = lax.axis_index("s")
    zr = acc.shape[0] // _NS
    pltpu.sync_copy(z_h, acc.at[pl.ds(s * zr, zr)])
    pltpu.sync_copy(xh_hbm.at[c, pl.ds(s * zr, zr)], xs.at[pl.ds(s * zr, zr)])
    if with_ef:
        pltpu.sync_copy(z_f, facc.at[pl.ds(s * zr, zr)])
    plsc.subcore_barrier()

    B = [bufs[3 * b:3 * b + 3] for b in range(NB)]
    for j in range(2):
        iv, rv, gsm = B[j % NB]
        pltpu.sync_copy(sdr.at[s, j], iv)
        pltpu.async_copy(xs.at[iv.at[0]], rv, gsm)

    def body(g, carry):
        for b in range(NB):
            iv, rv, gsm = B[b]
            ssm = bufs[3 * NB + b]
            j = NB * g + b
            pltpu.make_async_copy(xs.at[iv.at[0]], rv, gsm).wait()
            pltpu.async_copy(rv, acc.at[iv.at[1]], ssm, add=True)
            if with_ef:
                @pl.when((j % 2) == c)
                def _():
                    @pl.when(j >= 2)
                    def _():
                        pltpu.make_async_copy(ef_v, facc.at[iv.at[1]], efsm).wait()
                    pltpu.sync_copy(efr.at[s, j], ef_v)
                    pltpu.async_copy(ef_v, facc.at[iv.at[1]], efsm, add=True)
            b2 = (b + 2) % NB
            iv2, rv2, gsm2 = B[b2]
            ssm2 = bufs[3 * NB + b2]

            @pl.when(j + 2 < NCH)
            def _():
                @pl.when(j + 2 >= NB)
                def _():
                    pltpu.make_async_copy(rv2, acc.at[iv2.at[1]], ssm2).wait()
                pltpu.sync_copy(sdr.at[s, j + 2], iv2)
                pltpu.async_copy(xs.at[iv2.at[0]], rv2, gsm2)

        return carry

    lax.fori_loop(0, NCH // NB, body, 0)
    for j in range(NCH - NB, NCH):
        iv, rv, gsm = B[j % NB]
        ssm = bufs[3 * NB + j % NB]
        pltpu.make_async_copy(rv, acc.at[iv.at[1]], ssm).wait()
    if with_ef:
        pltpu.make_async_copy(ef_v, facc.at[B[0][0].at[1]], efsm).wait()
    plsc.subcore_barrier()
    pltpu.sync_copy(acc.at[pl.ds(s * zr, zr)], a_out.at[c, pl.ds(s * zr, zr)])
    if with_ef:
        pltpu.sync_copy(facc.at[pl.ds(s * zr, zr)],
                        f_out.at[c, pl.ds(s * zr, zr)])


_NB = 3


def _make_sc_seg1(NP, NCH, DH, DEA):
    def body(*args):
        _sc_seg_body(NCH, _NB, True, args)

    scr = [
        pltpu.VMEM_SHARED((NP, DH), jnp.float32),
        pltpu.VMEM_SHARED((NP, DEA), jnp.float32),
        pltpu.VMEM_SHARED((NP, DH), jnp.float32),
        pltpu.VMEM((_CH, DEA), jnp.float32),
        pltpu.SemaphoreType.DMA,
    ]
    for _ in range(_NB):
        scr += [pltpu.VMEM((2, _CH), jnp.int32),
                pltpu.VMEM((_CH, DH), jnp.float32),
                pltpu.SemaphoreType.DMA]
    scr += [pltpu.SemaphoreType.DMA] * _NB
    return pl.kernel(
        body,
        out_type=[jax.ShapeDtypeStruct((_NC, NP, DH), jnp.float32),
                  jax.ShapeDtypeStruct((_NC, NP, DEA), jnp.float32)],
        mesh=_mesh(),
        scratch_types=scr,
        compiler_params=pltpu.CompilerParams(use_tc_tiling_on_sc=False),
        interpret=_INTERPRET,
    )


def _make_sc_seg2(NP, NCH, DH):
    def body(*args):
        _sc_seg_body(NCH, _NB, False, args)

    scr = [
        pltpu.VMEM_SHARED((NP, DH), jnp.float32),
        pltpu.VMEM_SHARED((NP, DH), jnp.float32),
    ]
    for _ in range(_NB):
        scr += [pltpu.VMEM((2, _CH), jnp.int32),
                pltpu.VMEM((_CH, DH), jnp.float32),
                pltpu.SemaphoreType.DMA]
    scr += [pltpu.SemaphoreType.DMA] * _NB
    return pl.kernel(
        body,
        out_type=jax.ShapeDtypeStruct((_NC, NP, DH), jnp.float32),
        mesh=_mesh(),
        scratch_types=scr,
        compiler_params=pltpu.CompilerParams(use_tc_tiling_on_sc=False),
        interpret=_INTERPRET,
    )


def _tc1_body(a_ref, f_ref, w1n_ref, w1ea_ref, w2ea_ref, x1_ref, efw2_ref):
    a = jnp.concatenate([a_ref[0], a_ref[1]], axis=-1)
    f = f_ref[0] + f_ref[1]
    x1 = (jnp.dot(a, w1n_ref[...], preferred_element_type=jnp.float32,
                  precision=lax.Precision.HIGHEST)
          + jnp.dot(f, w1ea_ref[...], preferred_element_type=jnp.float32,
                    precision=lax.Precision.HIGHEST))
    x1 = jnp.maximum(x1, 0.0)
    DH = x1_ref.shape[-1]
    x1_ref[0] = x1[:, :DH]
    x1_ref[1] = x1[:, DH:]
    efw2_ref[...] = jnp.dot(f, w2ea_ref[...], preferred_element_type=jnp.float32,
                            precision=lax.Precision.HIGHEST)


def _make_tc1(NP, D, DEA, H):
    BM = NP // 8
    DH = D // 2
    grid = (8,)
    return pl.pallas_call(
        _tc1_body,
        grid=grid,
        in_specs=[
            pl.BlockSpec((_NC, BM, DH), lambda i: (0, i, 0)),
            pl.BlockSpec((_NC, BM, DEA), lambda i: (0, i, 0)),
            pl.BlockSpec((D, H), lambda i: (0, 0)),
            pl.BlockSpec((DEA, H), lambda i: (0, 0)),
            pl.BlockSpec((DEA, H), lambda i: (0, 0)),
        ],
        out_specs=[
            pl.BlockSpec((_NC, BM, H // 2), lambda i: (0, i, 0)),
            pl.BlockSpec((BM, H), lambda i: (i, 0)),
        ],
        out_shape=[jax.ShapeDtypeStruct((_NC, NP, H // 2), jnp.float32),
                   jax.ShapeDtypeStruct((NP, H), jnp.float32)],
        interpret=_INTERPRET,
    )


def _tc2_body(N, BM, b_ref, efw2_ref, w2n_ref, out_ref):
    i = pl.program_id(0)
    b = jnp.concatenate([b_ref[0], b_ref[1]], axis=-1)
    out2 = (jnp.dot(b, w2n_ref[...], preferred_element_type=jnp.float32,
                    precision=lax.Precision.HIGHEST)
            + efw2_ref[...])
    rmax = jnp.max(out2, axis=1)
    rmin = jnp.min(out2, axis=1)
    rows = lax.broadcasted_iota(jnp.int32, (BM,), 0) + i * BM
    m = ((rmax != rmin) & (rows < N)).astype(jnp.float32)
    part = jnp.sum(out2 * m[:, None], axis=0)

    @pl.when(i == 0)
    def _():
        out_ref[...] = jnp.zeros_like(out_ref)

    out_ref[...] += part[None, :]


def _make_tc2(N, NP, H):
    BM = NP // 8
    grid = (8,)
    return pl.pallas_call(
        functools.partial(_tc2_body, N, BM),
        grid=grid,
        in_specs=[
            pl.BlockSpec((_NC, BM, H // 2), lambda i: (0, i, 0)),
            pl.BlockSpec((BM, H), lambda i: (i, 0)),
            pl.BlockSpec((H, H), lambda i: (0, 0)),
        ],
        out_specs=pl.BlockSpec((1, H), lambda i: (0, 0)),
        out_shape=jax.ShapeDtypeStruct((1, H), jnp.float32),
        interpret=_INTERPRET,
    )


def kernel(node_feature, edge_index, edge_feature, W1n, b1n, W1e, b1e,
           W2n, b2n, W2e, b2e):
    N, D = node_feature.shape
    E, DE = edge_feature.shape
    H = W1n.shape[1]
    DH = D // 2
    DEA = 32

    src = edge_index[0].astype(jnp.int32)
    dst = edge_index[1].astype(jnp.int32)

    NP = ((N + 1 + 127) // 128) * 128
    epw = _NS * _CH * _NB
    Ep = ((E + epw - 1) // epw) * epw
    pad = Ep - E
    pad_dst = N + jnp.arange(pad, dtype=jnp.int32) % (NP - N)
    src = jnp.concatenate([src, jnp.zeros((pad,), jnp.int32)])
    dst = jnp.concatenate([dst, pad_dst])
    NCH = Ep // (_NS * _CH)
    sdr = jnp.stack([src.reshape(_NS, NCH, _CH), dst.reshape(_NS, NCH, _CH)],
                    axis=2)

    ef_aug = (jnp.zeros((Ep, DEA), jnp.float32)
              .at[:E, :DE].set(edge_feature)
              .at[:E, DE].set(1.0))
    efr = ef_aug.reshape(_NS, NCH, _CH, DEA)

    zrows = NP // _NS
    z_h = jnp.zeros((zrows, DH), jnp.float32)
    z_f = jnp.zeros((zrows, DEA), jnp.float32)

    xpad = jnp.zeros((NP, D), jnp.float32).at[:N].set(node_feature)
    xh = jnp.stack([xpad[:, :DH], xpad[:, DH:]])

    W1ea = jnp.zeros((DEA, H), jnp.float32).at[:DE].set(W1e).at[DE].set(b1n + b1e)
    W2ea = jnp.zeros((DEA, H), jnp.float32).at[:DE].set(W2e).at[DE].set(b2n + b2e)

    A, F = _make_sc_seg1(NP, NCH, DH, DEA)(xh, sdr, efr, z_h, z_f)
    X1h, EFW2 = _make_tc1(NP, D, DEA, H)(A, F, W1n, W1ea, W2ea)
    B = _make_sc_seg2(NP, NCH, H // 2)(X1h, sdr, z_h)
    pooled = _make_tc2(N, NP, H)(B, EFW2, W2n)
    return pooled.reshape(H)
